# baseline (speedup 1.0000x reference)
import functools

import jax
import jax.numpy as jnp
from jax import lax
from jax.experimental import pallas as pl
from jax.experimental.pallas import tpu as pltpu
from jax.experimental.pallas import tpu_sc as plsc

H = 128
E = 65536
T = 262144
SCN = 8

NC = 2
NS = 16
NW = NC * NS


def _silu(v):
    return v / (1.0 + jnp.exp(-v))


BE = 2048


def _edge_prep(x, rbf, W_rbf, W_ji, b_ji, W_kj, b_kj):
    def body(x_ref, rbf_ref, wr_ref, wji_ref, bji_ref, wkj_ref, bkj_ref,
             xji_ref, xkj_ref):
        xb = x_ref[...]
        rh = jnp.dot(rbf_ref[...], wr_ref[...],
                     preferred_element_type=jnp.float32)
        xji_ref[...] = _silu(jnp.dot(xb, wji_ref[...],
                                     preferred_element_type=jnp.float32)
                             + bji_ref[...])
        xkj_ref[...] = _silu(jnp.dot(xb, wkj_ref[...],
                                     preferred_element_type=jnp.float32)
                             + bkj_ref[...]) * rh

    grid = (E // BE,)
    return pl.pallas_call(
        body,
        grid=grid,
        in_specs=[
            pl.BlockSpec((BE, H), lambda i: (i, 0)),
            pl.BlockSpec((BE, 6), lambda i: (i, 0)),
            pl.BlockSpec((6, H), lambda i: (0, 0)),
            pl.BlockSpec((H, H), lambda i: (0, 0)),
            pl.BlockSpec((1, H), lambda i: (0, 0)),
            pl.BlockSpec((H, H), lambda i: (0, 0)),
            pl.BlockSpec((1, H), lambda i: (0, 0)),
        ],
        out_specs=[
            pl.BlockSpec((BE, H), lambda i: (i, 0)),
            pl.BlockSpec((BE, H), lambda i: (i, 0)),
        ],
        out_shape=[
            jax.ShapeDtypeStruct((E, H), jnp.float32),
            jax.ShapeDtypeStruct((E, H), jnp.float32),
        ],
    )(x, rbf, W_rbf, W_ji, b_ji.reshape(1, H), W_kj, b_kj.reshape(1, H))


GB = 128
G_STEPS = T // NW // GB


def _sc_gather(table, idx):
    mesh = plsc.VectorSubcoreMesh(core_axis_name="c", subcore_axis_name="s")

    @functools.partial(
        pl.kernel,
        out_type=jax.ShapeDtypeStruct((T, H), jnp.float32),
        mesh=mesh,
        scratch_types=[
            pltpu.VMEM((GB,), jnp.int32),
            pltpu.VMEM((GB, H), jnp.float32),
            pltpu.SemaphoreType.DMA,
        ],
        compiler_params=pltpu.CompilerParams(needs_layout_passes=False),
    )
    def k(table_hbm, idx_hbm, out_hbm, idx_v, rows_v, sem):
        wid = lax.axis_index("c") * NS + lax.axis_index("s")
        base = wid * (T // NW)

        def body(i, _):
            off = base + i * GB
            pltpu.sync_copy(idx_hbm.at[pl.ds(off, GB)], idx_v)
            pltpu.async_copy(table_hbm.at[idx_v], rows_v, sem).wait()
            pltpu.sync_copy(rows_v, out_hbm.at[pl.ds(off, GB)])
            return 0

        lax.fori_loop(0, G_STEPS, body, 0, unroll=False)

    return k(table, idx)


BT = 1024


def _triplet(t, sbf, angle2d, W_sbf, sel_W, W_bil_t):
    def body(t_ref, sbf_ref, ang_ref, wsbf_ref, selw_ref, wbil_ref, y_ref):
        tb = t_ref[...].astype(jnp.bfloat16)
        c = jnp.dot(sbf_ref[...], wsbf_ref[...],
                    preferred_element_type=jnp.float32)
        sel = jnp.floor(ang_ref[...] / 3.141593 * SCN).astype(jnp.int32)
        xsel = jnp.zeros(tb.shape, jnp.float32)
        for s in range(SCN):
            xs = jnp.dot(tb, selw_ref[s], preferred_element_type=jnp.float32)
            xsel = xsel + jnp.where(sel == s, 1.0, 0.0) * xs
        y = jnp.zeros(tb.shape, jnp.float32)
        for j in range(SCN):
            scaled = (xsel * c[:, j][:, None]).astype(jnp.bfloat16)
            y = y + lax.dot_general(
                scaled, wbil_ref[j], (((1,), (1,)), ((), ())),
                preferred_element_type=jnp.float32)
        y_ref[...] = y

    grid = (T // BT,)
    return pl.pallas_call(
        body,
        grid=grid,
        in_specs=[
            pl.BlockSpec((BT, H), lambda i: (i, 0)),
            pl.BlockSpec((BT, 42), lambda i: (i, 0)),
            pl.BlockSpec((BT, 1), lambda i: (i, 0)),
            pl.BlockSpec((42, SCN), lambda i: (0, 0)),
            pl.BlockSpec((SCN, H, H), lambda i: (0, 0, 0)),
            pl.BlockSpec((SCN, H, H), lambda i: (0, 0, 0)),
        ],
        out_specs=pl.BlockSpec((BT, H), lambda i: (i, 0)),
        out_shape=jax.ShapeDtypeStruct((T, H), jnp.float32),
    )(t, sbf, angle2d, W_sbf, sel_W.astype(jnp.bfloat16),
      W_bil_t.astype(jnp.bfloat16))


CR = 8192
NCHUNK = E // CR
CPC = NCHUNK // NC
TRASH = CR
ACC_ROWS = CR + 64
ZR = 129
FL = 128
TPW = T // NS
IDXC = 128
NFMAX = TPW // FL + 2
CAP = NFMAX * FL


def _sc_scatter_add(y, idx):
    mesh = plsc.VectorSubcoreMesh(core_axis_name="c", subcore_axis_name="s")

    @functools.partial(
        pl.kernel,
        out_type=jax.ShapeDtypeStruct((E, H), jnp.float32),
        mesh=mesh,
        scratch_types=[
            pltpu.VMEM((ZR, H), jnp.float32),
            pltpu.VMEM((IDXC,), jnp.int32),
            pltpu.VMEM((CAP,), jnp.int32),
            pltpu.VMEM((FL,), jnp.int32),
            pltpu.VMEM((FL,), jnp.int32),
            pltpu.VMEM((FL, H), jnp.float32),
            pltpu.VMEM_SHARED((ACC_ROWS, H), jnp.float32),
            pltpu.SemaphoreType.DMA,
        ],
        compiler_params=pltpu.CompilerParams(needs_layout_passes=False),
    )
    def k(y_hbm, idx_hbm, out_hbm, zbuf, idx_v, pkflat, wsm, dsm,
          rows_v, accum, sem):
        cid = lax.axis_index("c")
        tid = lax.axis_index("s")
        tbase = tid * TPW

        def zb(i, _):
            r = i // (H // 16)
            cc = (i % (H // 16)) * 16
            zbuf[r, pl.ds(cc, 16)] = jnp.zeros((16,), jnp.float32)
            return 0

        lax.fori_loop(0, ZR * (H // 16), zb, 0, unroll=False)

        for ck in range(CPC):
            chunk = cid * CPC + ck
            cbase = chunk * CR

            for j in range(4):
                pltpu.sync_copy(
                    zbuf, accum.at[pl.ds((tid * 4 + j) * ZR, ZR)])
            plsc.subcore_barrier()

            def outer(o, off):
                pltpu.sync_copy(
                    idx_hbm.at[pl.ds(tbase + o * IDXC, IDXC)], idx_v)

                def inner(kk, off):
                    iv = idx_v[pl.ds(kk * 16, 16)]
                    m = (iv >> 13) == chunk
                    w = (tbase + o * IDXC + kk * 16
                         + lax.iota(jnp.int32, 16))
                    dloc = iv & (CR - 1)
                    pk = (w << 14) | dloc
                    _, pks = plsc.sort_key_val(1 - m.astype(jnp.int32), pk)
                    pkflat[pl.ds(off, 16)] = pks
                    return off + jnp.sum(m.astype(jnp.int32))

                return lax.fori_loop(0, IDXC // 16, inner, off, unroll=False)

            off = lax.fori_loop(0, TPW // IDXC, outer, jnp.int32(0),
                                unroll=False)

            def pad(p, _):
                pkflat[pl.ds(off + p * 16, 16)] = jnp.full(
                    (16,), TRASH + tid, jnp.int32)
                return 0

            lax.fori_loop(0, FL // 16, pad, 0, unroll=False)

            nf = (off + FL - 1) // FL

            def flush(f, _):
                def cp(p, _):
                    pk = pkflat[pl.ds(f * FL + p * 16, 16)]
                    wsm[pl.ds(p * 16, 16)] = lax.shift_right_logical(pk, 14)
                    dsm[pl.ds(p * 16, 16)] = pk & 16383
                    return 0

                lax.fori_loop(0, FL // 16, cp, 0, unroll=False)
                pltpu.async_copy(y_hbm.at[wsm], rows_v, sem).wait()
                pltpu.sync_copy(rows_v, accum.at[dsm], add=True)
                return 0

            lax.fori_loop(0, nf, flush, 0, unroll=False)
            plsc.subcore_barrier()

            pltpu.sync_copy(accum.at[pl.ds(tid * (CR // NS), CR // NS)],
                            out_hbm.at[pl.ds(cbase + tid * (CR // NS),
                                             CR // NS)])
            plsc.subcore_barrier()

    return k(y, idx)


def _final_mlp(x, x_ji, acc, W_bs1, b_bs1, W_bs2, b_bs2, W_lin, b_lin,
               W_as1a, b_as1a, W_as1b, b_as1b, W_as2a, b_as2a, W_as2b,
               b_as2b):
    def body(x_ref, xji_ref, acc_ref, w1_ref, c1_ref, w2_ref, c2_ref,
             wl_ref, cl_ref, wa_ref, ca_ref, wb_ref, cb_ref, wc_ref, cc_ref,
             wd_ref, cd_ref, out_ref):
        def mm(v, w_ref, b_ref):
            return jnp.dot(v, w_ref[...],
                           preferred_element_type=jnp.float32) + b_ref[...]

        h = xji_ref[...] + acc_ref[...]
        h = h + _silu(mm(_silu(mm(h, w1_ref, c1_ref)), w2_ref, c2_ref))
        h = _silu(mm(h, wl_ref, cl_ref)) + x_ref[...]
        h = h + _silu(mm(_silu(mm(h, wa_ref, ca_ref)), wb_ref, cb_ref))
        h = h + _silu(mm(_silu(mm(h, wc_ref, cc_ref)), wd_ref, cd_ref))
        out_ref[...] = h

    grid = (E // BE,)
    row = pl.BlockSpec((BE, H), lambda i: (i, 0))
    wspec = pl.BlockSpec((H, H), lambda i: (0, 0))
    bspec = pl.BlockSpec((1, H), lambda i: (0, 0))
    return pl.pallas_call(
        body,
        grid=grid,
        in_specs=[row, row, row] + [wspec, bspec] * 7,
        out_specs=row,
        out_shape=jax.ShapeDtypeStruct((E, H), jnp.float32),
    )(x, x_ji, acc, W_bs1, b_bs1.reshape(1, H), W_bs2, b_bs2.reshape(1, H),
      W_lin, b_lin.reshape(1, H), W_as1a, b_as1a.reshape(1, H),
      W_as1b, b_as1b.reshape(1, H), W_as2a, b_as2a.reshape(1, H),
      W_as2b, b_as2b.reshape(1, H))


def kernel(x, rbf, sbf, idx_kj, idx_ji, angle, W_rbf, W_sbf, W_ji, b_ji,
           W_kj, b_kj, sel_W, W_bil, W_bs1, b_bs1, W_bs2, b_bs2, W_lin,
           b_lin, W_as1a, b_as1a, W_as1b, b_as1b, W_as2a, b_as2a, W_as2b,
           b_as2b):
    x_ji, x_kj_edge = _edge_prep(x, rbf, W_rbf, W_ji, b_ji, W_kj, b_kj)
    t = _sc_gather(x_kj_edge, idx_kj)
    W_bil_t = jnp.transpose(W_bil, (1, 0, 2))
    y = _triplet(t, sbf, angle.reshape(T, 1), W_sbf, sel_W, W_bil_t)
    acc = _sc_scatter_add(y, idx_ji)
    return _final_mlp(x, x_ji, acc, W_bs1, b_bs1, W_bs2, b_bs2, W_lin,
                      b_lin, W_as1a, b_as1a, W_as1b, b_as1b, W_as2a,
                      b_as2a, W_as2b, b_as2b)


# speedup vs baseline: 1.0217x; 1.0217x over previous
import functools

import jax
import jax.numpy as jnp
from jax import lax
from jax.experimental import pallas as pl
from jax.experimental.pallas import tpu as pltpu
from jax.experimental.pallas import tpu_sc as plsc

H = 128
E = 65536
T = 262144
SCN = 8

NC = 2
NS = 16
NW = NC * NS


def _silu(v):
    return v / (1.0 + jnp.exp(-v))


BE = 2048


def _edge_prep(x, rbf, W_rbf, W_ji, b_ji, W_kj, b_kj):
    def body(x_ref, rbf_ref, wr_ref, wji_ref, bji_ref, wkj_ref, bkj_ref,
             xji_ref, xkj_ref):
        xb = x_ref[...]
        rh = jnp.dot(rbf_ref[...], wr_ref[...],
                     preferred_element_type=jnp.float32)
        xji_ref[...] = _silu(jnp.dot(xb, wji_ref[...],
                                     preferred_element_type=jnp.float32)
                             + bji_ref[...])
        xkj_ref[...] = _silu(jnp.dot(xb, wkj_ref[...],
                                     preferred_element_type=jnp.float32)
                             + bkj_ref[...]) * rh

    grid = (E // BE,)
    return pl.pallas_call(
        body,
        grid=grid,
        in_specs=[
            pl.BlockSpec((BE, H), lambda i: (i, 0)),
            pl.BlockSpec((BE, 6), lambda i: (i, 0)),
            pl.BlockSpec((6, H), lambda i: (0, 0)),
            pl.BlockSpec((H, H), lambda i: (0, 0)),
            pl.BlockSpec((1, H), lambda i: (0, 0)),
            pl.BlockSpec((H, H), lambda i: (0, 0)),
            pl.BlockSpec((1, H), lambda i: (0, 0)),
        ],
        out_specs=[
            pl.BlockSpec((BE, H), lambda i: (i, 0)),
            pl.BlockSpec((BE, H), lambda i: (i, 0)),
        ],
        out_shape=[
            jax.ShapeDtypeStruct((E, H), jnp.float32),
            jax.ShapeDtypeStruct((E, H), jnp.float32),
        ],
    )(x, rbf, W_rbf, W_ji, b_ji.reshape(1, H), W_kj, b_kj.reshape(1, H))


GB = 128
G_STEPS = T // NW // GB


def _sc_gather(table, idx):
    mesh = plsc.VectorSubcoreMesh(core_axis_name="c", subcore_axis_name="s")

    @functools.partial(
        pl.kernel,
        out_type=jax.ShapeDtypeStruct((T, H), jnp.float32),
        mesh=mesh,
        scratch_types=[
            pltpu.VMEM((GB,), jnp.int32),
            pltpu.VMEM((GB, H), jnp.float32),
            pltpu.SemaphoreType.DMA,
        ],
        compiler_params=pltpu.CompilerParams(needs_layout_passes=False),
    )
    def k(table_hbm, idx_hbm, out_hbm, idx_v, rows_v, sem):
        wid = lax.axis_index("c") * NS + lax.axis_index("s")
        base = wid * (T // NW)

        def body(i, _):
            off = base + i * GB
            pltpu.sync_copy(idx_hbm.at[pl.ds(off, GB)], idx_v)
            pltpu.async_copy(table_hbm.at[idx_v], rows_v, sem).wait()
            pltpu.sync_copy(rows_v, out_hbm.at[pl.ds(off, GB)])
            return 0

        lax.fori_loop(0, G_STEPS, body, 0, unroll=False)

    return k(table, idx)


BT = 1024


def _triplet(t, sbf_t, angle2d, W_sbf, sel_W, W_bil_t):
    def body(t_ref, sbf_ref, ang_ref, wsbf_ref, selw_ref, wbil_ref, y_ref):
        tb = t_ref[...].astype(jnp.bfloat16)
        c = lax.dot_general(sbf_ref[...], wsbf_ref[...],
                            (((0,), (0,)), ((), ())),
                            preferred_element_type=jnp.float32)
        a8 = lax.dot_general(ang_ref[...], jnp.ones((1, SCN), jnp.float32),
                             (((0,), (0,)), ((), ())),
                             preferred_element_type=jnp.float32)
        sel8 = jnp.floor(a8 / 3.141593 * SCN).astype(jnp.int32)
        onehot = (sel8 == lax.broadcasted_iota(jnp.int32, (BT, SCN), 1))
        xsel = jnp.zeros(tb.shape, jnp.float32)
        for s in range(SCN):
            xs = jnp.dot(tb, selw_ref[s], preferred_element_type=jnp.float32)
            xsel = xsel + onehot[:, s].astype(jnp.float32)[:, None] * xs
        y = jnp.zeros(tb.shape, jnp.float32)
        for j in range(SCN):
            scaled = (xsel * c[:, j][:, None]).astype(jnp.bfloat16)
            y = y + lax.dot_general(
                scaled, wbil_ref[j], (((1,), (1,)), ((), ())),
                preferred_element_type=jnp.float32)
        y_ref[...] = y

    grid = (T // BT,)
    return pl.pallas_call(
        body,
        grid=grid,
        in_specs=[
            pl.BlockSpec((BT, H), lambda i: (i, 0)),
            pl.BlockSpec((42, BT), lambda i: (0, i)),
            pl.BlockSpec((1, BT), lambda i: (0, i)),
            pl.BlockSpec((42, SCN), lambda i: (0, 0)),
            pl.BlockSpec((SCN, H, H), lambda i: (0, 0, 0)),
            pl.BlockSpec((SCN, H, H), lambda i: (0, 0, 0)),
        ],
        out_specs=pl.BlockSpec((BT, H), lambda i: (i, 0)),
        out_shape=jax.ShapeDtypeStruct((T, H), jnp.float32),
    )(t, sbf_t, angle2d, W_sbf, sel_W.astype(jnp.bfloat16),
      W_bil_t.astype(jnp.bfloat16))


CR = 8192
NCHUNK = E // CR
CPC = NCHUNK // NC
TRASH = CR
ACC_ROWS = CR + 64
ZR = 129
FL = 128
TPW = T // NS
IDXC = 128
NFMAX = TPW // FL + 2
CAP = NFMAX * FL


def _sc_scatter_add(y, idx):
    mesh = plsc.VectorSubcoreMesh(core_axis_name="c", subcore_axis_name="s")

    @functools.partial(
        pl.kernel,
        out_type=jax.ShapeDtypeStruct((E, H), jnp.float32),
        mesh=mesh,
        scratch_types=[
            pltpu.VMEM((ZR, H), jnp.float32),
            pltpu.VMEM((IDXC,), jnp.int32),
            pltpu.VMEM((CAP,), jnp.int32),
            pltpu.VMEM((FL,), jnp.int32),
            pltpu.VMEM((FL,), jnp.int32),
            pltpu.VMEM((FL, H), jnp.float32),
            pltpu.VMEM_SHARED((ACC_ROWS, H), jnp.float32),
            pltpu.SemaphoreType.DMA,
        ],
        compiler_params=pltpu.CompilerParams(needs_layout_passes=False),
    )
    def k(y_hbm, idx_hbm, out_hbm, zbuf, idx_v, pkflat, wsm, dsm,
          rows_v, accum, sem):
        cid = lax.axis_index("c")
        tid = lax.axis_index("s")
        tbase = tid * TPW

        def zb(i, _):
            r = i // (H // 16)
            cc = (i % (H // 16)) * 16
            zbuf[r, pl.ds(cc, 16)] = jnp.zeros((16,), jnp.float32)
            return 0

        lax.fori_loop(0, ZR * (H // 16), zb, 0, unroll=False)

        for ck in range(CPC):
            chunk = cid * CPC + ck
            cbase = chunk * CR

            for j in range(4):
                pltpu.sync_copy(
                    zbuf, accum.at[pl.ds((tid * 4 + j) * ZR, ZR)])
            plsc.subcore_barrier()

            def outer(o, off):
                pltpu.sync_copy(
                    idx_hbm.at[pl.ds(tbase + o * IDXC, IDXC)], idx_v)

                def inner(kk, off):
                    iv = idx_v[pl.ds(kk * 16, 16)]
                    m = (iv >> 13) == chunk
                    w = (tbase + o * IDXC + kk * 16
                         + lax.iota(jnp.int32, 16))
                    dloc = iv & (CR - 1)
                    pk = (w << 14) | dloc
                    _, pks = plsc.sort_key_val(1 - m.astype(jnp.int32), pk)
                    pkflat[pl.ds(off, 16)] = pks
                    return off + jnp.sum(m.astype(jnp.int32))

                return lax.fori_loop(0, IDXC // 16, inner, off, unroll=False)

            off = lax.fori_loop(0, TPW // IDXC, outer, jnp.int32(0),
                                unroll=False)

            def pad(p, _):
                pkflat[pl.ds(off + p * 16, 16)] = jnp.full(
                    (16,), TRASH + tid, jnp.int32)
                return 0

            lax.fori_loop(0, FL // 16, pad, 0, unroll=False)

            nf = (off + FL - 1) // FL

            def flush(f, _):
                def cp(p, _):
                    pk = pkflat[pl.ds(f * FL + p * 16, 16)]
                    wsm[pl.ds(p * 16, 16)] = lax.shift_right_logical(pk, 14)
                    dsm[pl.ds(p * 16, 16)] = pk & 16383
                    return 0

                lax.fori_loop(0, FL // 16, cp, 0, unroll=False)
                pltpu.async_copy(y_hbm.at[wsm], rows_v, sem).wait()
                pltpu.sync_copy(rows_v, accum.at[dsm], add=True)
                return 0

            lax.fori_loop(0, nf, flush, 0, unroll=False)
            plsc.subcore_barrier()

            pltpu.sync_copy(accum.at[pl.ds(tid * (CR // NS), CR // NS)],
                            out_hbm.at[pl.ds(cbase + tid * (CR // NS),
                                             CR // NS)])
            plsc.subcore_barrier()

    return k(y, idx)


def _final_mlp(x, x_ji, acc, W_bs1, b_bs1, W_bs2, b_bs2, W_lin, b_lin,
               W_as1a, b_as1a, W_as1b, b_as1b, W_as2a, b_as2a, W_as2b,
               b_as2b):
    def body(x_ref, xji_ref, acc_ref, w1_ref, c1_ref, w2_ref, c2_ref,
             wl_ref, cl_ref, wa_ref, ca_ref, wb_ref, cb_ref, wc_ref, cc_ref,
             wd_ref, cd_ref, out_ref):
        def mm(v, w_ref, b_ref):
            return jnp.dot(v, w_ref[...],
                           preferred_element_type=jnp.float32) + b_ref[...]

        h = xji_ref[...] + acc_ref[...]
        h = h + _silu(mm(_silu(mm(h, w1_ref, c1_ref)), w2_ref, c2_ref))
        h = _silu(mm(h, wl_ref, cl_ref)) + x_ref[...]
        h = h + _silu(mm(_silu(mm(h, wa_ref, ca_ref)), wb_ref, cb_ref))
        h = h + _silu(mm(_silu(mm(h, wc_ref, cc_ref)), wd_ref, cd_ref))
        out_ref[...] = h

    grid = (E // BE,)
    row = pl.BlockSpec((BE, H), lambda i: (i, 0))
    wspec = pl.BlockSpec((H, H), lambda i: (0, 0))
    bspec = pl.BlockSpec((1, H), lambda i: (0, 0))
    return pl.pallas_call(
        body,
        grid=grid,
        in_specs=[row, row, row] + [wspec, bspec] * 7,
        out_specs=row,
        out_shape=jax.ShapeDtypeStruct((E, H), jnp.float32),
    )(x, x_ji, acc, W_bs1, b_bs1.reshape(1, H), W_bs2, b_bs2.reshape(1, H),
      W_lin, b_lin.reshape(1, H), W_as1a, b_as1a.reshape(1, H),
      W_as1b, b_as1b.reshape(1, H), W_as2a, b_as2a.reshape(1, H),
      W_as2b, b_as2b.reshape(1, H))


def kernel(x, rbf, sbf, idx_kj, idx_ji, angle, W_rbf, W_sbf, W_ji, b_ji,
           W_kj, b_kj, sel_W, W_bil, W_bs1, b_bs1, W_bs2, b_bs2, W_lin,
           b_lin, W_as1a, b_as1a, W_as1b, b_as1b, W_as2a, b_as2a, W_as2b,
           b_as2b):
    x_ji, x_kj_edge = _edge_prep(x, rbf, W_rbf, W_ji, b_ji, W_kj, b_kj)
    t = _sc_gather(x_kj_edge, idx_kj)
    W_bil_t = jnp.transpose(W_bil, (1, 0, 2))
    y = _triplet(t, sbf.T, angle.reshape(1, T), W_sbf, sel_W, W_bil_t)
    acc = _sc_scatter_add(y, idx_ji)
    return _final_mlp(x, x_ji, acc, W_bs1, b_bs1, W_bs2, b_bs2, W_lin,
                      b_lin, W_as1a, b_as1a, W_as1b, b_as1b, W_as2a,
                      b_as2a, W_as2b, b_as2b)
